# baseline (speedup 1.0000x reference)
import jax
import jax.numpy as jnp
from jax import lax
from jax.experimental import pallas as pl
from jax.experimental.pallas import tpu as pltpu
from jax.experimental.pallas import tpu_sc as plsc

B = 16384
NWORKERS = 32
RPW = B // NWORKERS


def _sc_body(y_hbm, w_hbm, c_hbm, t_hbm, ivy_hbm, ivw_hbm, ivc_hbm,
             yidx_hbm, widx_hbm, cidx_hbm,
             oy_hbm, ow_hbm, oc_hbm,
             y_v, w_v, c_v, ivy_v, ivw_v, ivc_v, idx_v, t_v):
    wid = lax.axis_index("s") * 2 + lax.axis_index("c")

    pltpu.sync_copy(t_hbm, t_v)
    tvec = t_v[...]
    ones = jnp.full((16,), 1, jnp.int32)
    zeros = jnp.full((16,), 0, jnp.int32)
    kv = zeros
    for i in range(1, 10):
        kv = kv + jnp.where(tvec >= 10.0 * i, ones, zeros)

    pltpu.sync_copy(yidx_hbm, idx_v.at[pl.ds(0, 64)])
    pltpu.sync_copy(widx_hbm, idx_v.at[pl.ds(64, 32)])
    pltpu.sync_copy(cidx_hbm, idx_v.at[pl.ds(96, 16)])

    lanes = lax.iota(jnp.int32, 16)
    scols_y = [idx_v[pl.ds(16 * ch, 16)] for ch in range(4)]
    scols_w = [idx_v[pl.ds(64, 16)], idx_v[pl.ds(80, 16)]]
    scols_c = [idx_v[pl.ds(96, 16)]]
    jcols = [lanes + 16 * ch for ch in range(4)]

    def do_array(in_hbm, iv_hbm, out_hbm, data_v, iv_v, scols, rows_per_chunk):
        nchunks = RPW // rows_per_chunk
        base0 = wid * RPW
        ngroups = len(scols)

        def chunk_body(i, carry):
            base = base0 + i * rows_per_chunk
            pltpu.sync_copy(in_hbm.at[pl.ds(base, rows_per_chunk)], data_v)
            pltpu.sync_copy(iv_hbm.at[pl.ds(base, rows_per_chunk)], iv_v)

            def row_body(r, c2):
                rv = jnp.full((16,), r, dtype=jnp.int32)
                for ch in range(ngroups):
                    v = plsc.load_gather(iv_v, [rv, jcols[ch], kv])
                    v = jnp.maximum(v, 0.0)
                    plsc.store_scatter(data_v, [rv, scols[ch]], v)
                return c2

            lax.fori_loop(0, rows_per_chunk, row_body, 0)
            pltpu.sync_copy(data_v, out_hbm.at[pl.ds(base, rows_per_chunk)])
            return carry

        lax.fori_loop(0, nchunks, chunk_body, 0)

    do_array(y_hbm, ivy_hbm, oy_hbm, y_v, ivy_v, scols_y, 16)
    do_array(w_hbm, ivw_hbm, ow_hbm, w_v, ivw_v, scols_w, 32)
    do_array(c_hbm, ivc_hbm, oc_hbm, c_v, ivc_v, scols_c, 64)


def kernel(y, w, c, t, iv_y, iv_w, iv_c, y_idx, w_idx, c_idx):
    t16 = jnp.broadcast_to(jnp.reshape(t, (1,)), (16,)).astype(jnp.float32)

    mesh = plsc.VectorSubcoreMesh(core_axis_name="c", subcore_axis_name="s")
    f = pl.kernel(
        _sc_body,
        mesh=mesh,
        out_type=[
            jax.ShapeDtypeStruct((B, 512), jnp.float32),
            jax.ShapeDtypeStruct((B, 256), jnp.float32),
            jax.ShapeDtypeStruct((B, 128), jnp.float32),
        ],
        scratch_types=[
            pltpu.VMEM((16, 512), jnp.float32),
            pltpu.VMEM((32, 256), jnp.float32),
            pltpu.VMEM((64, 128), jnp.float32),
            pltpu.VMEM((16, 64, 10), jnp.float32),
            pltpu.VMEM((32, 32, 10), jnp.float32),
            pltpu.VMEM((64, 16, 10), jnp.float32),
            pltpu.VMEM((112,), jnp.int32),
            pltpu.VMEM((16,), jnp.float32),
        ],
        compiler_params=pltpu.CompilerParams(
            needs_layout_passes=False, use_tc_tiling_on_sc=False),
    )
    oy, ow, oc = f(y, w, c, t16, iv_y, iv_w, iv_c,
                   y_idx.astype(jnp.int32), w_idx.astype(jnp.int32),
                   c_idx.astype(jnp.int32))
    return (oy, ow, oc)


# speedup vs baseline: 14.2385x; 14.2385x over previous
import jax
import jax.numpy as jnp
from jax import lax
from jax.experimental import pallas as pl
from jax.experimental.pallas import tpu as pltpu
from jax.experimental.pallas import tpu_sc as plsc

B = 16384
NWORKERS = 32
RPW = B // NWORKERS
CR = 32
NCHUNKS = RPW // CR


def _sc_body(y_hbm, w_hbm, c_hbm, ivy_hbm, ivw_hbm, ivc_hbm,
             yidx_hbm, widx_hbm, cidx_hbm,
             oy_hbm, ow_hbm, oc_hbm,
             y_v0, y_v1, w_v0, w_v1, c_v0, c_v1,
             ivy_v0, ivy_v1, ivw_v0, ivw_v1, ivc_v0, ivc_v1,
             idx_v,
             *sems):
    wid = lax.axis_index("s") * 2 + lax.axis_index("c")
    base0 = wid * RPW

    pltpu.sync_copy(yidx_hbm, idx_v.at[pl.ds(0, 64)])
    pltpu.sync_copy(widx_hbm, idx_v.at[pl.ds(64, 32)])
    pltpu.sync_copy(cidx_hbm, idx_v.at[pl.ds(96, 16)])

    lanes = lax.iota(jnp.int32, 16)
    scols_y = [idx_v[pl.ds(16 * ch, 16)] for ch in range(4)]
    scols_w = [idx_v[pl.ds(64, 16)], idx_v[pl.ds(80, 16)]]
    scols_c = [idx_v[pl.ds(96, 16)]]
    jcols = [lanes + 16 * ch for ch in range(4)]

    arrays = [
        (y_hbm, ivy_hbm, oy_hbm, (y_v0, y_v1), (ivy_v0, ivy_v1), scols_y),
        (w_hbm, ivw_hbm, ow_hbm, (w_v0, w_v1), (ivw_v0, ivw_v1), scols_w),
        (c_hbm, ivc_hbm, oc_hbm, (c_v0, c_v1), (ivc_v0, ivc_v1), scols_c),
    ]
    in_sems = sems[0:6]
    out_sems = sems[6:12]

    def compute_chunk(data_v, iv_v, scols):
        ngroups = len(scols)

        def row_body(r, c2):
            rv = jnp.full((16,), r, dtype=jnp.int32)
            for ch in range(ngroups):
                v = plsc.load_gather(iv_v, [rv, jcols[ch]])
                v = jnp.maximum(v, 0.0)
                plsc.store_scatter(data_v, [rv, scols[ch]], v)
            return c2

        lax.fori_loop(0, CR, row_body, 0)

    tail = []
    for a, (in_hbm, iv_hbm, out_hbm, dbufs, vbufs, scols) in enumerate(arrays):
        pend_in = [None, None]
        pend_out = [None, None]

        def start_in(i, a=a, in_hbm=in_hbm, iv_hbm=iv_hbm, dbufs=dbufs,
                     vbufs=vbufs, pend_in=pend_in, pend_out=pend_out):
            p = i & 1
            if pend_out[p] is not None:
                pend_out[p].wait()
                pend_out[p] = None
            rows = pl.ds(base0 + i * CR, CR)
            hd = pltpu.async_copy(in_hbm.at[rows], dbufs[p], in_sems[2 * a + p])
            hv = pltpu.async_copy(iv_hbm.at[rows], vbufs[p], in_sems[2 * a + p])
            pend_in[p] = (hd, hv)

        start_in(0)
        for i in range(NCHUNKS):
            p = i & 1
            if i + 1 < NCHUNKS:
                start_in(i + 1)
            hd, hv = pend_in[p]
            hd.wait()
            hv.wait()
            compute_chunk(dbufs[p], vbufs[p], scols)
            pend_out[p] = pltpu.async_copy(
                dbufs[p], out_hbm.at[pl.ds(base0 + i * CR, CR)],
                out_sems[2 * a + p])
        tail.append(pend_out[0])
        tail.append(pend_out[1])

    for h in tail:
        if h is not None:
            h.wait()


def kernel(y, w, c, t, iv_y, iv_w, iv_c, y_idx, w_idx, c_idx):
    starts = jnp.arange(1, 10, dtype=jnp.float32) * 10.0
    k = jnp.sum((t >= starts).astype(jnp.int32))
    ivy2 = lax.dynamic_index_in_dim(iv_y, k, axis=2, keepdims=False)
    ivw2 = lax.dynamic_index_in_dim(iv_w, k, axis=2, keepdims=False)
    ivc2 = lax.dynamic_index_in_dim(iv_c, k, axis=2, keepdims=False)

    mesh = plsc.VectorSubcoreMesh(core_axis_name="c", subcore_axis_name="s")
    f = pl.kernel(
        _sc_body,
        mesh=mesh,
        out_type=[
            jax.ShapeDtypeStruct((B, 512), jnp.float32),
            jax.ShapeDtypeStruct((B, 256), jnp.float32),
            jax.ShapeDtypeStruct((B, 128), jnp.float32),
        ],
        scratch_types=(
            [pltpu.VMEM((CR, 512), jnp.float32)] * 2
            + [pltpu.VMEM((CR, 256), jnp.float32)] * 2
            + [pltpu.VMEM((CR, 128), jnp.float32)] * 2
            + [pltpu.VMEM((CR, 64), jnp.float32)] * 2
            + [pltpu.VMEM((CR, 32), jnp.float32)] * 2
            + [pltpu.VMEM((CR, 16), jnp.float32)] * 2
            + [pltpu.VMEM((112,), jnp.int32)]
            + [pltpu.SemaphoreType.DMA] * 12
        ),
        compiler_params=pltpu.CompilerParams(needs_layout_passes=False),
    )
    oy, ow, oc = f(y, w, c, ivy2, ivw2, ivc2,
                   y_idx.astype(jnp.int32), w_idx.astype(jnp.int32),
                   c_idx.astype(jnp.int32))
    return (oy, ow, oc)


# speedup vs baseline: 14.7571x; 1.0364x over previous
import jax
import jax.numpy as jnp
from jax import lax
from jax.experimental import pallas as pl
from jax.experimental.pallas import tpu as pltpu
from jax.experimental.pallas import tpu_sc as plsc

B = 16384
NWORKERS = 32
RPW = B // NWORKERS
CRY = 64
CRW = 32


def _sc_body(y_hbm, w_hbm, c_hbm, ivy_hbm, ivw_hbm, ivc_hbm,
             yidx_hbm, widx_hbm, cidx_hbm,
             oy_hbm, ow_hbm, oc_hbm,
             y_v0, y_v1, w_v0, w_v1, c_v0, c_v1,
             ivy_v0, ivy_v1, ivw_v0, ivw_v1, ivc_v0, ivc_v1,
             idx_v,
             *sems):
    wid = lax.axis_index("s") * 2 + lax.axis_index("c")
    base0 = wid * RPW

    arrays = [
        (y_hbm, ivy_hbm, oy_hbm, (y_v0, y_v1), (ivy_v0, ivy_v1), CRY, 4),
        (w_hbm, ivw_hbm, ow_hbm, (w_v0, w_v1), (ivw_v0, ivw_v1), CRW, 2),
        (c_hbm, ivc_hbm, oc_hbm, (c_v0, c_v1), (ivc_v0, ivc_v1), CRW, 1),
    ]
    in_sems = sems[0:6]
    out_sems = sems[6:12]

    pend_in = [[None, None] for _ in arrays]
    pend_out = [[None, None] for _ in arrays]

    def start_in(a, i):
        in_hbm, iv_hbm, _, dbufs, vbufs, cr, _ = arrays[a]
        p = i & 1
        if pend_out[a][p] is not None:
            pend_out[a][p].wait()
            pend_out[a][p] = None
        rows = pl.ds(base0 + i * cr, cr)
        hd = pltpu.async_copy(in_hbm.at[rows], dbufs[p], in_sems[2 * a + p])
        hv = pltpu.async_copy(iv_hbm.at[rows], vbufs[p], in_sems[2 * a + p])
        pend_in[a][p] = (hd, hv)

    for a in range(3):
        start_in(a, 0)

    pltpu.sync_copy(yidx_hbm, idx_v.at[pl.ds(0, 64)])
    pltpu.sync_copy(widx_hbm, idx_v.at[pl.ds(64, 32)])
    pltpu.sync_copy(cidx_hbm, idx_v.at[pl.ds(96, 16)])

    lanes = lax.iota(jnp.int32, 16)
    scols = [
        [idx_v[pl.ds(16 * ch, 16)] for ch in range(4)],
        [idx_v[pl.ds(64, 16)], idx_v[pl.ds(80, 16)]],
        [idx_v[pl.ds(96, 16)]],
    ]
    jcols = [lanes + 16 * ch for ch in range(4)]

    def compute_chunk(data_v, iv_v, cols, cr):
        ngroups = len(cols)

        def rows_body(i2, c2):
            r = i2 * 2
            for dr in range(2):
                rv = jnp.full((16,), r + dr, dtype=jnp.int32)
                for ch in range(ngroups):
                    v = plsc.load_gather(iv_v, [rv, jcols[ch]])
                    v = jnp.maximum(v, 0.0)
                    plsc.store_scatter(data_v, [rv, cols[ch]], v)
            return c2

        lax.fori_loop(0, cr // 2, rows_body, 0)

    for a, (in_hbm, iv_hbm, out_hbm, dbufs, vbufs, cr, _ng) in enumerate(arrays):
        nchunks = RPW // cr
        for i in range(nchunks):
            p = i & 1
            if i + 1 < nchunks:
                start_in(a, i + 1)
            hd, hv = pend_in[a][p]
            hd.wait()
            hv.wait()
            compute_chunk(dbufs[p], vbufs[p], scols[a], cr)
            pend_out[a][p] = pltpu.async_copy(
                dbufs[p], out_hbm.at[pl.ds(base0 + i * cr, cr)],
                out_sems[2 * a + p])

    for a in range(3):
        for h in pend_out[a]:
            if h is not None:
                h.wait()


def kernel(y, w, c, t, iv_y, iv_w, iv_c, y_idx, w_idx, c_idx):
    starts = jnp.arange(1, 10, dtype=jnp.float32) * 10.0
    k = jnp.sum((t >= starts).astype(jnp.int32))
    ivy2 = lax.dynamic_index_in_dim(iv_y, k, axis=2, keepdims=False)
    ivw2 = lax.dynamic_index_in_dim(iv_w, k, axis=2, keepdims=False)
    ivc2 = lax.dynamic_index_in_dim(iv_c, k, axis=2, keepdims=False)

    mesh = plsc.VectorSubcoreMesh(core_axis_name="c", subcore_axis_name="s")
    f = pl.kernel(
        _sc_body,
        mesh=mesh,
        out_type=[
            jax.ShapeDtypeStruct((B, 512), jnp.float32),
            jax.ShapeDtypeStruct((B, 256), jnp.float32),
            jax.ShapeDtypeStruct((B, 128), jnp.float32),
        ],
        scratch_types=(
            [pltpu.VMEM((CRY, 512), jnp.float32)] * 2
            + [pltpu.VMEM((CRW, 256), jnp.float32)] * 2
            + [pltpu.VMEM((CRW, 128), jnp.float32)] * 2
            + [pltpu.VMEM((CRY, 64), jnp.float32)] * 2
            + [pltpu.VMEM((CRW, 32), jnp.float32)] * 2
            + [pltpu.VMEM((CRW, 16), jnp.float32)] * 2
            + [pltpu.VMEM((112,), jnp.int32)]
            + [pltpu.SemaphoreType.DMA] * 12
        ),
        compiler_params=pltpu.CompilerParams(needs_layout_passes=False),
    )
    oy, ow, oc = f(y, w, c, ivy2, ivw2, ivc2,
                   y_idx.astype(jnp.int32), w_idx.astype(jnp.int32),
                   c_idx.astype(jnp.int32))
    return (oy, ow, oc)


# speedup vs baseline: 16.0599x; 1.0883x over previous
import jax
import jax.numpy as jnp
from jax import lax
from jax.experimental import pallas as pl
from jax.experimental.pallas import tpu as pltpu
from jax.experimental.pallas import tpu_sc as plsc

B = 16384
NWORKERS = 32
RPW = B // NWORKERS
CRY = 64
CRW = 32
CRC = 64
IVC = 128


def _sc_body(y_hbm, w_hbm, c_hbm, t_hbm, ivy_hbm, ivw_hbm, ivc_hbm,
             yidx_hbm, widx_hbm, cidx_hbm,
             oy_hbm, ow_hbm, oc_hbm,
             y_v0, y_v1, w_v0, w_v1, c_v0, c_v1,
             ivy_v0, ivy_v1, ivw_v0, ivw_v1, ivc_v0, ivc_v1,
             idx_v, t_v,
             *sems):
    wid = lax.axis_index("s") * 2 + lax.axis_index("c")
    base0 = wid * RPW

    pltpu.sync_copy(t_hbm, t_v)
    tvec = t_v[...]
    ones = jnp.full((16,), 1, jnp.int32)
    zeros = jnp.full((16,), 0, jnp.int32)
    kv = zeros
    for i in range(1, 10):
        kv = kv + jnp.where(tvec >= 10.0 * i, ones, zeros)
    k = jnp.max(kv)

    arrays = [
        (y_hbm, ivy_hbm, oy_hbm, (y_v0, y_v1), (ivy_v0, ivy_v1), CRY, 4),
        (w_hbm, ivw_hbm, ow_hbm, (w_v0, w_v1), (ivw_v0, ivw_v1), CRW, 2),
        (c_hbm, ivc_hbm, oc_hbm, (c_v0, c_v1), (ivc_v0, ivc_v1), CRC, 1),
    ]
    in_sems = sems[0:6]
    iv_sems = sems[6:12]
    out_sems = sems[12:18]

    pend_in = [[None, None] for _ in arrays]
    pend_iv = [[None, None] for _ in arrays]
    pend_out = [[None, None] for _ in arrays]

    def start_in(a, i):
        in_hbm, _, _, dbufs, _, cr, _ = arrays[a]
        p = i & 1
        if pend_out[a][p] is not None:
            pend_out[a][p].wait()
            pend_out[a][p] = None
        rows = pl.ds(base0 + i * cr, cr)
        pend_in[a][p] = pltpu.async_copy(
            in_hbm.at[rows], dbufs[p], in_sems[2 * a + p])

    def start_iv(a, ii):
        _, iv_hbm, _, _, vbufs, _, _ = arrays[a]
        q = ii & 1
        cols = pl.ds(base0 + ii * IVC, IVC)
        pend_iv[a][q] = pltpu.async_copy(
            iv_hbm.at[k, slice(None), cols], vbufs[q], iv_sems[2 * a + q])

    for a in range(3):
        start_in(a, 0)
        start_iv(a, 0)

    pltpu.sync_copy(yidx_hbm, idx_v.at[pl.ds(0, 64)])
    pltpu.sync_copy(widx_hbm, idx_v.at[pl.ds(64, 32)])
    pltpu.sync_copy(cidx_hbm, idx_v.at[pl.ds(96, 16)])

    lanes = lax.iota(jnp.int32, 16)
    scols = [
        [idx_v[pl.ds(16 * ch, 16)] for ch in range(4)],
        [idx_v[pl.ds(64, 16)], idx_v[pl.ds(80, 16)]],
        [idx_v[pl.ds(96, 16)]],
    ]
    jcols = [lanes + 16 * ch for ch in range(4)]

    def compute_chunk(data_v, iv_v, cols, cr, rcol0):
        ngroups = len(cols)

        def rows_body(i2, c2):
            r = i2 * 2
            for dr in range(2):
                rv = jnp.full((16,), r + dr, dtype=jnp.int32)
                rc = jnp.full((16,), rcol0 + r + dr, dtype=jnp.int32)
                for ch in range(ngroups):
                    v = plsc.load_gather(iv_v, [jcols[ch], rc])
                    v = jnp.maximum(v, 0.0)
                    plsc.store_scatter(data_v, [rv, cols[ch]], v)
            return c2

        lax.fori_loop(0, cr // 2, rows_body, 0)

    for a, (in_hbm, iv_hbm, out_hbm, dbufs, vbufs, cr, _ng) in enumerate(arrays):
        nchunks = RPW // cr
        per_iv = IVC // cr
        for i in range(nchunks):
            p = i & 1
            if i + 1 < nchunks:
                start_in(a, i + 1)
            if i % per_iv == 0 and (i + per_iv) < nchunks:
                start_iv(a, i // per_iv + 1)
            hd = pend_in[a][p]
            hd.wait()
            q = (i // per_iv) & 1
            hv = pend_iv[a][q]
            if hv is not None:
                hv.wait()
                pend_iv[a][q] = None
            compute_chunk(dbufs[p], vbufs[q], scols[a], cr, (i % per_iv) * cr)
            pend_out[a][p] = pltpu.async_copy(
                dbufs[p], out_hbm.at[pl.ds(base0 + i * cr, cr)],
                out_sems[2 * a + p])

    for a in range(3):
        for h in pend_out[a]:
            if h is not None:
                h.wait()


def kernel(y, w, c, t, iv_y, iv_w, iv_c, y_idx, w_idx, c_idx):
    ivyT = jnp.transpose(iv_y, (2, 1, 0))
    ivwT = jnp.transpose(iv_w, (2, 1, 0))
    ivcT = jnp.transpose(iv_c, (2, 1, 0))
    t16 = jnp.broadcast_to(jnp.reshape(t, (1,)), (16,)).astype(jnp.float32)

    mesh = plsc.VectorSubcoreMesh(core_axis_name="c", subcore_axis_name="s")
    f = pl.kernel(
        _sc_body,
        mesh=mesh,
        out_type=[
            jax.ShapeDtypeStruct((B, 512), jnp.float32),
            jax.ShapeDtypeStruct((B, 256), jnp.float32),
            jax.ShapeDtypeStruct((B, 128), jnp.float32),
        ],
        scratch_types=(
            [pltpu.VMEM((CRY, 512), jnp.float32)] * 2
            + [pltpu.VMEM((CRW, 256), jnp.float32)] * 2
            + [pltpu.VMEM((CRC, 128), jnp.float32)] * 2
            + [pltpu.VMEM((64, IVC), jnp.float32)] * 2
            + [pltpu.VMEM((32, IVC), jnp.float32)] * 2
            + [pltpu.VMEM((16, IVC), jnp.float32)] * 2
            + [pltpu.VMEM((112,), jnp.int32)]
            + [pltpu.VMEM((16,), jnp.float32)]
            + [pltpu.SemaphoreType.DMA] * 18
        ),
        compiler_params=pltpu.CompilerParams(needs_layout_passes=False),
    )
    oy, ow, oc = f(y, w, c, t16, ivyT, ivwT, ivcT,
                   y_idx.astype(jnp.int32), w_idx.astype(jnp.int32),
                   c_idx.astype(jnp.int32))
    return (oy, ow, oc)
